# baseline (speedup 1.0000x reference)
import functools

import jax
import jax.numpy as jnp
from jax import lax
from jax.experimental import pallas as pl
from jax.experimental.pallas import tpu as pltpu
from jax.experimental.pallas import tpu_sc as plsc

_L = 16


def _merge16(a, b):
    rb = lax.rev(b, (0,))
    lo = jnp.minimum(a, rb)
    hi = jnp.maximum(a, rb)
    return lax.sort(lo), lax.sort(hi)


def _merge32(a0, a1, b0, b1):
    rb0 = lax.rev(b1, (0,))
    rb1 = lax.rev(b0, (0,))
    lo0 = jnp.minimum(a0, rb0)
    lo1 = jnp.minimum(a1, rb1)
    hi0 = jnp.maximum(a0, rb0)
    hi1 = jnp.maximum(a1, rb1)
    p0 = jnp.minimum(lo0, lo1)
    p1 = jnp.maximum(lo0, lo1)
    q0 = jnp.minimum(hi0, hi1)
    q1 = jnp.maximum(hi0, hi1)
    return lax.sort(p0), lax.sort(p1), lax.sort(q0), lax.sort(q1)


_N = 63


@functools.partial(
    pl.kernel,
    mesh=plsc.VectorSubcoreMesh(core_axis_name="c", subcore_axis_name="s"),
    out_type=jax.ShapeDtypeStruct((_N,), jnp.float32),
    scratch_types=[
        pltpu.VMEM((_N,), jnp.float32),
        pltpu.VMEM((4 * _L,), jnp.float32),
    ],
    compiler_params=pltpu.CompilerParams(needs_layout_passes=False),
)
def _sc_sigmoid_sort(raw_hbm, out_hbm, x_v, o_v):
    is_w0 = jnp.logical_and(lax.axis_index("c") == 0, lax.axis_index("s") == 0)

    @pl.when(is_w0)
    def _():
        pltpu.sync_copy(raw_hbm, x_v)
        v = [x_v[pl.ds(o, _L)] for o in (0, 16, 32, 47)]
        v = [1.0 / (1.0 + jnp.exp(-u)) for u in v]
        lane = lax.iota(jnp.int32, _L)
        v[3] = jnp.where(lane == 0, jnp.float32(jnp.inf), v[3])
        s = [lax.sort(u) for u in v]
        a0, a1 = _merge16(s[0], s[1])
        b0, b1 = _merge16(s[2], s[3])
        f = _merge32(a0, a1, b0, b1)
        for i in range(4):
            o_v[pl.ds(i * _L, _L)] = f[i]
        pltpu.sync_copy(o_v.at[pl.ds(0, _N)], out_hbm)


@jax.jit
def kernel(raw):
    return _sc_sigmoid_sort(raw)


# speedup vs baseline: 1.0985x; 1.0985x over previous
import functools

import jax
import jax.numpy as jnp
from jax import lax
from jax.experimental import pallas as pl
from jax.experimental.pallas import tpu as pltpu
from jax.experimental.pallas import tpu_sc as plsc

_L = 16


def _merge16(a, b):
    rb = lax.rev(b, (0,))
    lo = jnp.minimum(a, rb)
    hi = jnp.maximum(a, rb)
    return lax.sort(lo), lax.sort(hi)


def _merge32(a0, a1, b0, b1):
    rb0 = lax.rev(b1, (0,))
    rb1 = lax.rev(b0, (0,))
    lo0 = jnp.minimum(a0, rb0)
    lo1 = jnp.minimum(a1, rb1)
    hi0 = jnp.maximum(a0, rb0)
    hi1 = jnp.maximum(a1, rb1)
    p0 = jnp.minimum(lo0, lo1)
    p1 = jnp.maximum(lo0, lo1)
    q0 = jnp.minimum(hi0, hi1)
    q1 = jnp.maximum(hi0, hi1)
    return lax.sort(p0), lax.sort(p1), lax.sort(q0), lax.sort(q1)


_N = 63


@functools.partial(
    pl.kernel,
    mesh=plsc.VectorSubcoreMesh(
        core_axis_name="c", subcore_axis_name="s", num_cores=1, num_subcores=1
    ),
    out_type=jax.ShapeDtypeStruct((_N,), jnp.float32),
    scratch_types=[
        pltpu.VMEM((_N,), jnp.float32),
        pltpu.VMEM((4 * _L,), jnp.float32),
    ],
    compiler_params=pltpu.CompilerParams(needs_layout_passes=False),
)
def _sc_sigmoid_sort(raw_hbm, out_hbm, x_v, o_v):
    is_w0 = jnp.logical_and(lax.axis_index("c") == 0, lax.axis_index("s") == 0)

    @pl.when(is_w0)
    def _():
        pltpu.sync_copy(raw_hbm, x_v)
        v = [x_v[pl.ds(o, _L)] for o in (0, 16, 32, 47)]
        v = [1.0 / (1.0 + jnp.exp(-u)) for u in v]
        lane = lax.iota(jnp.int32, _L)
        v[3] = jnp.where(lane == 0, jnp.float32(jnp.inf), v[3])
        s = [lax.sort(u) for u in v]
        a0, a1 = _merge16(s[0], s[1])
        b0, b1 = _merge16(s[2], s[3])
        f = _merge32(a0, a1, b0, b1)
        for i in range(4):
            o_v[pl.ds(i * _L, _L)] = f[i]
        pltpu.sync_copy(o_v.at[pl.ds(0, _N)], out_hbm)


@jax.jit
def kernel(raw):
    return _sc_sigmoid_sort(raw)


# speedup vs baseline: 5.2708x; 4.7981x over previous
import jax
import jax.numpy as jnp
from jax import lax
from jax.experimental import pallas as pl


def _tc_body(x_ref, o_ref):
    s = 1.0 / (1.0 + jnp.exp(-x_ref[...]))
    lane = lax.broadcasted_iota(jnp.int32, (1, 64), 1)
    s = jnp.where(lane == 63, jnp.float32(jnp.inf), s)
    b = jnp.broadcast_to(s, (64, 64))
    a = b.T
    ii = lax.broadcasted_iota(jnp.int32, (64, 64), 0)
    jj = lax.broadcasted_iota(jnp.int32, (64, 64), 1)
    less = (b < a) | ((b == a) & (jj < ii))
    rank = jnp.sum(less.astype(jnp.int32), axis=1, keepdims=True)
    kk = lax.broadcasted_iota(jnp.int32, (64, 64), 1)
    m = jnp.where(rank == kk, a, 0.0)
    o_ref[...] = jnp.sum(m, axis=0, keepdims=True)


@jax.jit
def kernel(raw):
    x = jnp.pad(raw, (0, 1)).reshape(1, 64)
    out = pl.pallas_call(
        _tc_body,
        out_shape=jax.ShapeDtypeStruct((1, 64), jnp.float32),
    )(x)
    return out[0, :63]


# speedup vs baseline: 14.8290x; 2.8134x over previous
import jax
import jax.numpy as jnp
from jax import lax
from jax.experimental import pallas as pl

_N = 63


def _tc_body(x_ref, o_ref):
    s = 1.0 / (1.0 + jnp.exp(-x_ref[...]))
    b = jnp.broadcast_to(s, (_N, _N))
    a = b.T
    ii = lax.broadcasted_iota(jnp.int32, (_N, _N), 0)
    jj = lax.broadcasted_iota(jnp.int32, (_N, _N), 1)
    less = (b < a) | ((b == a) & (jj < ii))
    rank = jnp.sum(less.astype(jnp.int32), axis=1, keepdims=True)
    kk = lax.broadcasted_iota(jnp.int32, (_N, _N), 1)
    m = jnp.where(rank == kk, a, 0.0)
    o_ref[...] = jnp.sum(m, axis=0, keepdims=True)


@jax.jit
def kernel(raw):
    x = raw.reshape(1, _N)
    out = pl.pallas_call(
        _tc_body,
        out_shape=jax.ShapeDtypeStruct((1, _N), jnp.float32),
    )(x)
    return out.reshape(_N)
